# baseline (speedup 1.0000x reference)
import functools

import jax
import jax.numpy as jnp
from jax import lax
from jax.experimental import pallas as pl
from jax.experimental.pallas import tpu as pltpu
from jax.experimental.pallas import tpu_sc as plsc

BATCH = 16384
FACTORS = 64
NC = 2
NS = 16
NW = NC * NS
PER_W = BATCH // NW
CHUNK = 128
NCHUNK = PER_W // CHUNK
GROUPS = CHUNK // 16


def _bpr_body(user_hbm, item_i_hbm, item_j_hbm, ue_hbm, ie_hbm, ib_hbm,
              out_hbm,
              idx_u, idx_i, idx_j, ad_u, ad_i, ad_j,
              rows_u, rows_i, rows_j,
              bias_i, bias_j, out_v, sem):
    n_users = ue_hbm.shape[0] // FACTORS
    n_items = ie_hbm.shape[0] // FACTORS
    wid = lax.axis_index("s") * NC + lax.axis_index("c")
    base = wid * PER_W

    for c in range(NCHUNK):
        off = base + c * CHUNK
        pltpu.sync_copy(user_hbm.at[pl.ds(off, CHUNK)], idx_u.at[c])
        pltpu.sync_copy(item_i_hbm.at[pl.ds(off, CHUNK)], idx_i.at[c])
        pltpu.sync_copy(item_j_hbm.at[pl.ds(off, CHUNK)], idx_j.at[c])

    for c in range(NCHUNK):
        def build(f, carry):
            for g in range(GROUPS):
                sl = pl.ds(g * 16, 16)
                ad_u[f, sl] = idx_u[c, sl] + f * n_users
                ad_i[f, sl] = idx_i[c, sl] + f * n_items
                ad_j[f, sl] = idx_j[c, sl] + f * n_items
            return carry

        lax.fori_loop(0, FACTORS, build, 0)

        def fire(f, carry):
            pltpu.async_copy(ue_hbm.at[ad_u.at[f]], rows_u.at[f], sem)
            pltpu.async_copy(ie_hbm.at[ad_i.at[f]], rows_i.at[f], sem)
            pltpu.async_copy(ie_hbm.at[ad_j.at[f]], rows_j.at[f], sem)
            return carry

        lax.fori_loop(0, FACTORS, fire, 0)
        d4 = pltpu.async_copy(ib_hbm.at[idx_i.at[c]], bias_i, sem)
        d5 = pltpu.async_copy(ib_hbm.at[idx_j.at[c]], bias_j, sem)

        def drain(f, carry):
            pltpu.make_async_copy(ue_hbm.at[ad_u.at[f]], rows_u.at[f],
                                  sem).wait()
            pltpu.make_async_copy(ie_hbm.at[ad_i.at[f]], rows_i.at[f],
                                  sem).wait()
            pltpu.make_async_copy(ie_hbm.at[ad_j.at[f]], rows_j.at[f],
                                  sem).wait()
            return carry

        lax.fori_loop(0, FACTORS, drain, 0)
        d4.wait()
        d5.wait()

        def accum(f, accs):
            new = []
            for g in range(GROUPS):
                sl = pl.ds(g * 16, 16)
                u = rows_u[f, sl]
                vi = rows_i[f, sl]
                vj = rows_j[f, sl]
                new.append(accs[g] + u * (vi - vj))
            return tuple(new)

        zeros = jnp.zeros((16,), jnp.float32)
        accs = lax.fori_loop(0, FACTORS, accum, (zeros,) * GROUPS)
        for g in range(GROUPS):
            sl = pl.ds(g * 16, 16)
            out_v[pl.ds(c * CHUNK + g * 16, 16)] = (
                accs[g] + bias_i[sl] - bias_j[sl])

    pltpu.sync_copy(out_v, out_hbm.at[pl.ds(base, PER_W)])


_bpr_sc = functools.partial(
    pl.kernel,
    out_type=jax.ShapeDtypeStruct((BATCH,), jnp.float32),
    mesh=plsc.VectorSubcoreMesh(core_axis_name="c", subcore_axis_name="s"),
    compiler_params=pltpu.CompilerParams(needs_layout_passes=False),
    scratch_types=[
        pltpu.VMEM((NCHUNK, CHUNK), jnp.int32),
        pltpu.VMEM((NCHUNK, CHUNK), jnp.int32),
        pltpu.VMEM((NCHUNK, CHUNK), jnp.int32),
        pltpu.VMEM((FACTORS, CHUNK), jnp.int32),
        pltpu.VMEM((FACTORS, CHUNK), jnp.int32),
        pltpu.VMEM((FACTORS, CHUNK), jnp.int32),
        pltpu.VMEM((FACTORS, CHUNK), jnp.float32),
        pltpu.VMEM((FACTORS, CHUNK), jnp.float32),
        pltpu.VMEM((FACTORS, CHUNK), jnp.float32),
        pltpu.VMEM((CHUNK,), jnp.float32),
        pltpu.VMEM((CHUNK,), jnp.float32),
        pltpu.VMEM((PER_W,), jnp.float32),
        pltpu.SemaphoreType.DMA,
    ],
)(_bpr_body)


def kernel(user, item_i, item_j, user_embedding, item_embedding, item_bias):
    return _bpr_sc(user.astype(jnp.int32), item_i.astype(jnp.int32),
                   item_j.astype(jnp.int32),
                   user_embedding.T.reshape(-1),
                   item_embedding.T.reshape(-1),
                   item_bias.reshape(-1))


# speedup vs baseline: 3.1171x; 3.1171x over previous
import functools

import jax
import jax.numpy as jnp
from jax import lax
from jax.experimental import pallas as pl
from jax.experimental.pallas import tpu as pltpu
from jax.experimental.pallas import tpu_sc as plsc

BATCH = 16384
FACTORS = 64
NC = 2
NS = 16
NW = NC * NS
PER_W = BATCH // NW
CHUNK = 128
NCHUNK = PER_W // CHUNK
GROUPS = CHUNK // 16


def _bpr_body(user_hbm, item_i_hbm, item_j_hbm, ue_hbm, ie_hbm, ib_hbm,
              out_hbm,
              idx_u, idx_i, idx_j, pk_u, pk_i, pk_j,
              rows_u, rows_i, rows_j,
              bias_i, bias_j, tr, out_v, sem):
    wid = lax.axis_index("s") * NC + lax.axis_index("c")
    base = wid * PER_W
    iota = lax.iota(jnp.int32, 16)

    for c in range(NCHUNK):
        off = base + c * CHUNK
        pltpu.sync_copy(user_hbm.at[pl.ds(off, CHUNK)], idx_u.at[c])
        pltpu.sync_copy(item_i_hbm.at[pl.ds(off, CHUNK)], idx_i.at[c])
        pltpu.sync_copy(item_j_hbm.at[pl.ds(off, CHUNK)], idx_j.at[c])

    for c in range(NCHUNK):
        for s in range(CHUNK // 16):
            sl = pl.ds(s * 16, 16)
            pk_u[c, sl] = lax.shift_right_logical(idx_u[c, sl], 2)
            pk_i[c, sl] = lax.shift_right_logical(idx_i[c, sl], 2)
            pk_j[c, sl] = lax.shift_right_logical(idx_j[c, sl], 2)

    for c in range(NCHUNK):
        d1 = pltpu.async_copy(ue_hbm.at[pk_u.at[c]], rows_u, sem)
        d2 = pltpu.async_copy(ie_hbm.at[pk_i.at[c]], rows_i, sem)
        d3 = pltpu.async_copy(ie_hbm.at[pk_j.at[c]], rows_j, sem)
        d4 = pltpu.async_copy(ib_hbm.at[idx_i.at[c]], bias_i, sem)
        d5 = pltpu.async_copy(ib_hbm.at[idx_j.at[c]], bias_j, sem)
        d1.wait()
        d2.wait()
        d3.wait()
        d4.wait()
        d5.wait()

        def group_body(g, carry):
            gsl = pl.ds(g * 16, 16)
            hu_v = (idx_u[c, gsl] & 3) * (FACTORS // 2)
            hi_v = (idx_i[c, gsl] & 3) * (FACTORS // 2)
            hj_v = (idx_j[c, gsl] & 3) * (FACTORS // 2)
            for b16 in range(16):
                b = g * 16 + b16
                hu = hu_v[b16]
                hi = hi_v[b16]
                hj = hj_v[b16]
                acc = None
                for k in range(2):
                    ua, ub = plsc.unpack(
                        plsc.bitcast(rows_u[b, pl.ds(hu + k * 16, 16)],
                                     jnp.bfloat16),
                        format=plsc.PackFormat.INTERLEAVED)
                    ia, ib = plsc.unpack(
                        plsc.bitcast(rows_i[b, pl.ds(hi + k * 16, 16)],
                                     jnp.bfloat16),
                        format=plsc.PackFormat.INTERLEAVED)
                    ja, jb = plsc.unpack(
                        plsc.bitcast(rows_j[b, pl.ds(hj + k * 16, 16)],
                                     jnp.bfloat16),
                        format=plsc.PackFormat.INTERLEAVED)
                    p = ua * (ia - ja) + ub * (ib - jb)
                    acc = p if acc is None else acc + p
                tr[pl.ds(b16 * 16, 16)] = acc
            iota16 = iota * 16
            tot = plsc.load_gather(tr, [iota16])
            for cc in range(1, 16):
                tot = tot + plsc.load_gather(tr, [iota16 + cc])
            bi = bias_i[pl.ds(g * 16, 16)]
            bj = bias_j[pl.ds(g * 16, 16)]
            out_v[pl.ds(c * CHUNK + g * 16, 16)] = tot + bi - bj
            return carry

        lax.fori_loop(0, GROUPS, group_body, 0)

    pltpu.sync_copy(out_v, out_hbm.at[pl.ds(base, PER_W)])


_bpr_sc = functools.partial(
    pl.kernel,
    out_type=jax.ShapeDtypeStruct((BATCH,), jnp.float32),
    mesh=plsc.VectorSubcoreMesh(core_axis_name="c", subcore_axis_name="s"),
    compiler_params=pltpu.CompilerParams(needs_layout_passes=False),
    scratch_types=[
        pltpu.VMEM((NCHUNK, CHUNK), jnp.int32),
        pltpu.VMEM((NCHUNK, CHUNK), jnp.int32),
        pltpu.VMEM((NCHUNK, CHUNK), jnp.int32),
        pltpu.VMEM((NCHUNK, CHUNK), jnp.int32),
        pltpu.VMEM((NCHUNK, CHUNK), jnp.int32),
        pltpu.VMEM((NCHUNK, CHUNK), jnp.int32),
        pltpu.VMEM((CHUNK, 2 * FACTORS), jnp.int32),
        pltpu.VMEM((CHUNK, 2 * FACTORS), jnp.int32),
        pltpu.VMEM((CHUNK, 2 * FACTORS), jnp.int32),
        pltpu.VMEM((CHUNK,), jnp.float32),
        pltpu.VMEM((CHUNK,), jnp.float32),
        pltpu.VMEM((256,), jnp.float32),
        pltpu.VMEM((PER_W,), jnp.float32),
        pltpu.SemaphoreType.DMA,
    ],
)(_bpr_body)


def kernel(user, item_i, item_j, user_embedding, item_embedding, item_bias):
    def pack_bf16(t):
        n, d = t.shape
        t16 = t.astype(jnp.bfloat16).reshape(n, d // 2, 2)
        return jax.lax.bitcast_convert_type(t16, jnp.int32).reshape(n // 4, 2 * d)

    return _bpr_sc(user.astype(jnp.int32), item_i.astype(jnp.int32),
                   item_j.astype(jnp.int32),
                   pack_bf16(user_embedding),
                   pack_bf16(item_embedding),
                   item_bias.reshape(-1))


# speedup vs baseline: 9.1389x; 2.9318x over previous
import functools

import jax
import jax.numpy as jnp
from jax import lax
from jax.experimental import pallas as pl
from jax.experimental.pallas import tpu as pltpu
from jax.experimental.pallas import tpu_sc as plsc

BATCH = 16384
FACTORS = 64
NC = 2
NS = 16
NW = NC * NS
PER_W = BATCH // NW
CHUNK = 128
NCHUNK = PER_W // CHUNK
GROUPS = CHUNK // 16


def _bpr_body(user_hbm, item_i_hbm, item_j_hbm, ue_hbm, ie_hbm, ib_hbm,
              out_hbm,
              idx_u, idx_i, idx_j, rows_u, rows_i, rows_j,
              bias_i, bias_j, tr, out_v, sem):
    wid = lax.axis_index("s") * NC + lax.axis_index("c")
    base = wid * PER_W
    iota = lax.iota(jnp.int32, 16)

    for c in range(NCHUNK):
        off = base + c * CHUNK
        pltpu.sync_copy(user_hbm.at[pl.ds(off, CHUNK)], idx_u.at[c])
        pltpu.sync_copy(item_i_hbm.at[pl.ds(off, CHUNK)], idx_i.at[c])
        pltpu.sync_copy(item_j_hbm.at[pl.ds(off, CHUNK)], idx_j.at[c])

    for c in range(NCHUNK):
        d1 = pltpu.async_copy(ue_hbm.at[idx_u.at[c]], rows_u, sem)
        d2 = pltpu.async_copy(ie_hbm.at[idx_i.at[c]], rows_i, sem)
        d3 = pltpu.async_copy(ie_hbm.at[idx_j.at[c]], rows_j, sem)
        d4 = pltpu.async_copy(ib_hbm.at[idx_i.at[c]], bias_i, sem)
        d5 = pltpu.async_copy(ib_hbm.at[idx_j.at[c]], bias_j, sem)
        d1.wait()
        d2.wait()
        d3.wait()
        d4.wait()
        d5.wait()

        def group_body(g, carry):
            for b16 in range(16):
                b = g * 16 + b16
                acc = None
                for k in range(4):
                    sl = pl.ds(k * 16, 16)
                    p = rows_u[b, sl] * (rows_i[b, sl] - rows_j[b, sl])
                    acc = p if acc is None else acc + p
                tr[pl.ds(b16 * 16, 16)] = acc
            iota16 = iota * 16
            tot = plsc.load_gather(tr, [iota16])
            for cc in range(1, 16):
                tot = tot + plsc.load_gather(tr, [iota16 + cc])
            bi = bias_i[pl.ds(g * 16, 16)]
            bj = bias_j[pl.ds(g * 16, 16)]
            out_v[pl.ds(c * CHUNK + g * 16, 16)] = tot + bi - bj
            return carry

        lax.fori_loop(0, GROUPS, group_body, 0)

    pltpu.sync_copy(out_v, out_hbm.at[pl.ds(base, PER_W)])


_bpr_sc = functools.partial(
    pl.kernel,
    out_type=jax.ShapeDtypeStruct((BATCH,), jnp.float32),
    mesh=plsc.VectorSubcoreMesh(core_axis_name="c", subcore_axis_name="s"),
    compiler_params=pltpu.CompilerParams(needs_layout_passes=False,
                                         use_tc_tiling_on_sc=False),
    scratch_types=[
        pltpu.VMEM((NCHUNK, CHUNK), jnp.int32),
        pltpu.VMEM((NCHUNK, CHUNK), jnp.int32),
        pltpu.VMEM((NCHUNK, CHUNK), jnp.int32),
        pltpu.VMEM((CHUNK, FACTORS), jnp.float32),
        pltpu.VMEM((CHUNK, FACTORS), jnp.float32),
        pltpu.VMEM((CHUNK, FACTORS), jnp.float32),
        pltpu.VMEM((CHUNK,), jnp.float32),
        pltpu.VMEM((CHUNK,), jnp.float32),
        pltpu.VMEM((256,), jnp.float32),
        pltpu.VMEM((PER_W,), jnp.float32),
        pltpu.SemaphoreType.DMA,
    ],
)(_bpr_body)


def kernel(user, item_i, item_j, user_embedding, item_embedding, item_bias):
    return _bpr_sc(user.astype(jnp.int32), item_i.astype(jnp.int32),
                   item_j.astype(jnp.int32), user_embedding, item_embedding,
                   item_bias.reshape(-1))


# speedup vs baseline: 9.6949x; 1.0608x over previous
import functools

import jax
import jax.numpy as jnp
from jax import lax
from jax.experimental import pallas as pl
from jax.experimental.pallas import tpu as pltpu
from jax.experimental.pallas import tpu_sc as plsc

BATCH = 16384
FACTORS = 64
NC = 2
NS = 16
NW = NC * NS
PER_W = BATCH // NW
CHUNK = 128
NCHUNK = PER_W // CHUNK
GROUPS = CHUNK // 16


def _bpr_body(user_hbm, item_i_hbm, item_j_hbm, ue_hbm, ie_hbm, ib_hbm,
              out_hbm,
              idx_u, idx_i, idx_j, rows_u, rows_i, rows_j,
              bias_i, bias_j, tr, out_v, sem):
    wid = lax.axis_index("s") * NC + lax.axis_index("c")
    base = wid * PER_W
    iota = lax.iota(jnp.int32, 16)

    for c in range(NCHUNK):
        off = base + c * CHUNK
        pltpu.sync_copy(user_hbm.at[pl.ds(off, CHUNK)], idx_u.at[c])
        pltpu.sync_copy(item_i_hbm.at[pl.ds(off, CHUNK)], idx_i.at[c])
        pltpu.sync_copy(item_j_hbm.at[pl.ds(off, CHUNK)], idx_j.at[c])

    for c in range(NCHUNK):
        d1 = pltpu.async_copy(ue_hbm.at[idx_u.at[c]], rows_u, sem)
        d2 = pltpu.async_copy(ie_hbm.at[idx_i.at[c]], rows_i, sem)
        d3 = pltpu.async_copy(ie_hbm.at[idx_j.at[c]], rows_j, sem)
        d4 = pltpu.async_copy(ib_hbm.at[idx_i.at[c]], bias_i, sem)
        d5 = pltpu.async_copy(ib_hbm.at[idx_j.at[c]], bias_j, sem)
        d1.wait()
        d2.wait()
        d3.wait()
        d4.wait()
        d5.wait()

        def group_body(g, carry):
            for b16 in range(16):
                b = g * 16 + b16
                acc = None
                for k in range(4):
                    sl = pl.ds(k * 16, 16)
                    p = rows_u[b, sl] * (rows_i[b, sl] - rows_j[b, sl])
                    acc = p if acc is None else acc + p
                tr[pl.ds(b16 * 16, 16)] = acc
            iota16 = iota * 16
            tot = plsc.load_gather(tr, [iota16])
            for cc in range(1, 16):
                tot = tot + plsc.load_gather(tr, [iota16 + cc])
            bi = bias_i[pl.ds(g * 16, 16)]
            bj = bias_j[pl.ds(g * 16, 16)]
            out_v[pl.ds(c * CHUNK + g * 16, 16)] = tot + bi - bj
            return carry

        lax.fori_loop(0, GROUPS, group_body, 0)

    pltpu.sync_copy(out_v, out_hbm.at[pl.ds(base, PER_W)])


_bpr_sc = functools.partial(
    pl.kernel,
    out_type=jax.ShapeDtypeStruct((BATCH,), jnp.float32),
    mesh=plsc.VectorSubcoreMesh(core_axis_name="c", subcore_axis_name="s"),
    compiler_params=pltpu.CompilerParams(needs_layout_passes=False,
                                         use_tc_tiling_on_sc=True),
    scratch_types=[
        pltpu.VMEM((NCHUNK, CHUNK), jnp.int32),
        pltpu.VMEM((NCHUNK, CHUNK), jnp.int32),
        pltpu.VMEM((NCHUNK, CHUNK), jnp.int32),
        pltpu.VMEM((CHUNK, 2 * FACTORS), jnp.float32),
        pltpu.VMEM((CHUNK, 2 * FACTORS), jnp.float32),
        pltpu.VMEM((CHUNK, 2 * FACTORS), jnp.float32),
        pltpu.VMEM((CHUNK,), jnp.float32),
        pltpu.VMEM((CHUNK,), jnp.float32),
        pltpu.VMEM((256,), jnp.float32),
        pltpu.VMEM((PER_W,), jnp.float32),
        pltpu.SemaphoreType.DMA,
    ],
)(_bpr_body)


def kernel(user, item_i, item_j, user_embedding, item_embedding, item_bias):
    pad = ((0, 0), (0, FACTORS))
    return _bpr_sc(user.astype(jnp.int32), item_i.astype(jnp.int32),
                   item_j.astype(jnp.int32),
                   jnp.pad(user_embedding, pad),
                   jnp.pad(item_embedding, pad),
                   item_bias.reshape(-1))
